# Initial kernel scaffold; baseline (speedup 1.0000x reference)
import jax
import jax.numpy as jnp
from jax.experimental import pallas as pl


def kernel(wid, edge_index, root_ids, emb, Wz_w, Wz_b, Wr_w, Ur_w, Ur_b, Wh_w, Wh_b, Wg_w, Wg_b):
    raise NotImplementedError("write your pallas kernel here")


# speedup vs baseline: 3.2040x; 3.2040x over previous
import functools

import jax
import jax.numpy as jnp
from jax import lax
from jax.experimental import pallas as pl
from jax.experimental.pallas import tpu as pltpu
from jax.experimental.pallas import tpu_sc as plsc

_N = 10000
_NP = 10240
_E = 320000
_EU = _E // 2
_H = 128
_C = 80
_G = 5
_GC = _G * _C
_STRIPE = _NP // 16

_mesh = lambda: plsc.VectorSubcoreMesh(core_axis_name="c", subcore_axis_name="s")


def _f32(shape):
    return jax.ShapeDtypeStruct(shape, jnp.float32)


def _ceil8(n):
    return (n + 7) // 8 * 8


def _pack_idx(idx, n_slabs, rows_slab):
    r8 = _ceil8(rows_slab)
    a = idx.reshape(n_slabs, rows_slab, _C)
    a = jnp.pad(a, ((0, 0), (0, r8 - rows_slab), (0, 0)))
    return a.reshape(n_slabs * r8, _C)


def _make_gather1(n_rows, rows_tile, grp):
    per_tile = rows_tile * _C
    n_groups = rows_tile // grp
    gc = grp * _C
    r8 = _ceil8(rows_tile)
    assert per_tile * 32 == n_rows and n_groups * grp == rows_tile

    @functools.partial(
        pl.kernel, mesh=_mesh(),
        out_type=_f32((n_rows, _H)),
        scratch_types=[
            pltpu.VMEM((r8, _C), jnp.int32),
            pltpu.VMEM((gc, _H), jnp.float32),
            pltpu.SemaphoreType.DMA,
        ],
    )
    def k(table_hbm, idx_hbm, out_hbm, idx_v, rows_v, sem):
        w = lax.axis_index("s") * 2 + lax.axis_index("c")
        pltpu.sync_copy(idx_hbm.at[pl.ds(w * r8, r8)], idx_v)

        def group(g, carry):
            base = w * per_tile + g * gc
            hs = [
                pltpu.async_copy(
                    table_hbm.at[idx_v.at[g * grp + j]],
                    rows_v.at[pl.ds(j * _C, _C)], sem)
                for j in range(grp)
            ]
            for h_ in hs:
                h_.wait()
            pltpu.sync_copy(rows_v, out_hbm.at[pl.ds(base, gc)])
            return carry

        lax.fori_loop(0, n_groups, group, 0)

    return k


def _make_gather2():
    rows_tile = 250
    per_tile = rows_tile * _C
    n_groups = rows_tile // _G
    r8 = 256

    @functools.partial(
        pl.kernel, mesh=_mesh(),
        out_type=[_f32((_E, _H)), _f32((_E, _H))],
        scratch_types=[
            pltpu.VMEM((r8, _C), jnp.int32),
            pltpu.VMEM((_GC, _H), jnp.float32),
            pltpu.SemaphoreType.DMA,
        ],
    )
    def k(ta, tb, idx_hbm, oa, ob, idx_v, rows_v, sem):
        cid = lax.axis_index("c")
        sid = lax.axis_index("s")
        pltpu.sync_copy(idx_hbm.at[pl.ds(sid * r8, r8)], idx_v)

        def group(g, carry):
            base = sid * per_tile + g * _GC

            @pl.when(cid == 0)
            def _():
                hs = [
                    pltpu.async_copy(
                        ta.at[idx_v.at[g * _G + j]],
                        rows_v.at[pl.ds(j * _C, _C)], sem)
                    for j in range(_G)
                ]
                for h_ in hs:
                    h_.wait()
                pltpu.sync_copy(rows_v, oa.at[pl.ds(base, _GC)])

            @pl.when(cid == 1)
            def _():
                hs = [
                    pltpu.async_copy(
                        tb.at[idx_v.at[g * _G + j]],
                        rows_v.at[pl.ds(j * _C, _C)], sem)
                    for j in range(_G)
                ]
                for h_ in hs:
                    h_.wait()
                pltpu.sync_copy(rows_v, ob.at[pl.ds(base, _GC)])

            return carry

        lax.fori_loop(0, n_groups, group, 0)

    return k


def _make_gather_small():
    @functools.partial(
        pl.kernel, mesh=_mesh(),
        out_type=_f32((32, _H)),
        scratch_types=[
            pltpu.VMEM((32,), jnp.int32),
            pltpu.VMEM((32, _H), jnp.float32),
            pltpu.SemaphoreType.DMA,
        ],
    )
    def k(h_hbm, roots_hbm, out_hbm, idx_v, rows_v, sem):
        cid = lax.axis_index("c")
        sid = lax.axis_index("s")

        @pl.when((cid == 0) & (sid == 0))
        def _():
            pltpu.sync_copy(roots_hbm, idx_v)
            pltpu.async_copy(h_hbm.at[idx_v], rows_v, sem).wait()
            pltpu.sync_copy(rows_v, out_hbm)

    return k


def _make_segsum2():
    r8 = 256
    slab = 32
    n_slabs = r8 // slab
    sg = 2

    @functools.partial(
        pl.kernel, mesh=_mesh(),
        out_type=[_f32((_NP, _H)), _f32((_NP, _H))],
        scratch_types=[
            pltpu.VMEM((slab, _C), jnp.int32),
            pltpu.VMEM((sg * _C, _H), jnp.float32),
            pltpu.VMEM_SHARED((_NP, _H), jnp.float32),
            pltpu.SemaphoreType.DMA,
        ],
    )
    def k(m_hbm, rm_hbm, dst_hbm, zeros_hbm, nm_hbm, nrm_hbm,
          idx_v, rows_v, table, sem):
        cid = lax.axis_index("c")
        sid = lax.axis_index("s")
        pltpu.sync_copy(zeros_hbm.at[pl.ds(sid * _STRIPE, _STRIPE)],
                        table.at[pl.ds(sid * _STRIPE, _STRIPE)])
        plsc.subcore_barrier()

        def do_slab(s, carry):
            pltpu.sync_copy(dst_hbm.at[pl.ds(sid * r8 + s * slab, slab)],
                            idx_v)
            ng = jnp.where(s < n_slabs - 1, slab // sg, 26 // sg)

            def group(g, c2):
                base = sid * 20000 + s * slab * _C + g * sg * _C

                @pl.when(cid == 0)
                def _():
                    pltpu.sync_copy(m_hbm.at[pl.ds(base, sg * _C)], rows_v)

                @pl.when(cid == 1)
                def _():
                    pltpu.sync_copy(rm_hbm.at[pl.ds(base, sg * _C)], rows_v)

                for j in range(sg):
                    pltpu.sync_copy(rows_v.at[pl.ds(j * _C, _C)],
                                    table.at[idx_v.at[g * sg + j]], add=True)
                return c2

            lax.fori_loop(0, ng, group, 0)
            return carry

        lax.fori_loop(0, n_slabs, do_slab, 0)
        plsc.subcore_barrier()

        @pl.when(cid == 0)
        def _():
            pltpu.sync_copy(table.at[pl.ds(sid * _STRIPE, _STRIPE)],
                            nm_hbm.at[pl.ds(sid * _STRIPE, _STRIPE)])

        @pl.when(cid == 1)
        def _():
            pltpu.sync_copy(table.at[pl.ds(sid * _STRIPE, _STRIPE)],
                            nrm_hbm.at[pl.ds(sid * _STRIPE, _STRIPE)])

    return k


def _make_segsum_final():
    rows_tile = 125
    per_tile = rows_tile * _C
    r8 = 128
    sg = 2

    @functools.partial(
        pl.kernel, mesh=_mesh(),
        out_type=_f32((2, _NP, _H)),
        scratch_types=[
            pltpu.VMEM((r8, _C), jnp.int32),
            pltpu.VMEM((sg * _C, _H), jnp.float32),
            pltpu.VMEM_SHARED((_NP, _H), jnp.float32),
            pltpu.SemaphoreType.DMA,
        ],
    )
    def k(m_hbm, dst_hbm, zeros_hbm, p_hbm, idx_v, rows_v, table, sem):
        cid = lax.axis_index("c")
        sid = lax.axis_index("s")
        slab = cid * 16 + sid
        pltpu.sync_copy(zeros_hbm.at[pl.ds(sid * _STRIPE, _STRIPE)],
                        table.at[pl.ds(sid * _STRIPE, _STRIPE)])
        pltpu.sync_copy(dst_hbm.at[pl.ds(slab * r8, r8)], idx_v)
        plsc.subcore_barrier()

        def group(g, carry):
            base = slab * per_tile + g * sg * _C
            pltpu.sync_copy(m_hbm.at[pl.ds(base, sg * _C)], rows_v)
            for j in range(sg):
                pltpu.sync_copy(rows_v.at[pl.ds(j * _C, _C)],
                                table.at[idx_v.at[g * sg + j]], add=True)
            return carry

        lax.fori_loop(0, rows_tile // sg, group, 0)
        tbase = slab * per_tile + (rows_tile - 1) * _C
        pltpu.sync_copy(m_hbm.at[pl.ds(tbase, _C)],
                        rows_v.at[pl.ds(0, _C)])
        pltpu.sync_copy(rows_v.at[pl.ds(0, _C)],
                        table.at[idx_v.at[rows_tile - 1]], add=True)
        plsc.subcore_barrier()
        pltpu.sync_copy(table.at[pl.ds(sid * _STRIPE, _STRIPE)],
                        p_hbm.at[cid].at[pl.ds(sid * _STRIPE, _STRIPE)])

    return k


_BN = 1000
_BE = 1600
_NBE = _E // _BE
_HBE = _NBE // 2


def _nodeproj_body(x_ref, wz, wh, wr, wg, bz, bh, br, bg,
                   xz_o, xh_o, xr_o, xg_o):
    xv = x_ref[...]
    f32 = jnp.float32
    xz_o[...] = jnp.dot(xv, wz[...], preferred_element_type=f32) + bz[...]
    xh_o[...] = jnp.dot(xv, wh[...], preferred_element_type=f32) + bh[...]
    xr_o[...] = jnp.dot(xv, wr[...], preferred_element_type=f32) + br[...]
    xg_o[...] = jnp.dot(xv, wg[...], preferred_element_type=f32) + bg[...]


def _tc_nodeproj(x, wz1t, wh1t, wrt, wg1t, bz, bh, br, bg):
    n = x.shape[0]
    wspec = pl.BlockSpec((_H, _H), lambda i: (0, 0))
    bspec = pl.BlockSpec((1, _H), lambda i: (0, 0))
    return pl.pallas_call(
        _nodeproj_body,
        grid=(n // _BN,),
        in_specs=[pl.BlockSpec((_BN, _H), lambda i: (i, 0)),
                  wspec, wspec, wspec, wspec, bspec, bspec, bspec, bspec],
        out_specs=[pl.BlockSpec((_BN, _H), lambda i: (i, 0))] * 4,
        out_shape=[_f32((n, _H))] * 4,
    )(x, wz1t, wh1t, wrt, wg1t, bz, bh, br, bg)


def _gru0_body(xz, xh, xr, ur, m_o, rm_o):
    z = jax.nn.sigmoid(xz[...])
    mn = z * jnp.tanh(xh[...])
    r = jax.nn.sigmoid(
        xr[...] + jnp.dot(mn, ur[...], preferred_element_type=jnp.float32))
    m_o[...] = mn
    rm_o[...] = r * mn


def _tc_gru0(xz_src, xh_src, xr_dst, urt):
    espec = pl.BlockSpec((_BE, _H), lambda i: (i, 0))
    wspec = pl.BlockSpec((_H, _H), lambda i: (0, 0))
    return pl.pallas_call(
        _gru0_body,
        grid=(_NBE,),
        in_specs=[espec, espec, espec, wspec],
        out_specs=[espec, espec],
        out_shape=[_f32((_E, _H))] * 2,
    )(xz_src, xh_src, xr_dst, urt)


def _gru_body(xz, xh, xr, gm, grm, mrev, rmrev, wz2, wh2, ur, m_o, rm_o):
    f32 = jnp.float32
    s = gm[...] - mrev[...]
    a = grm[...] - rmrev[...]
    z = jax.nn.sigmoid(
        xz[...] + jnp.dot(s, wz2[...], preferred_element_type=f32))
    mn = (1.0 - z) * s + z * jnp.tanh(
        xh[...] + jnp.dot(a, wh2[...], preferred_element_type=f32))
    r = jax.nn.sigmoid(
        xr[...] + jnp.dot(mn, ur[...], preferred_element_type=f32))
    m_o[...] = mn
    rm_o[...] = r * mn


def _tc_gru(xz_src, xh_src, xr_dst, g_m, g_rm, m, rm, wz2t, wh2t, urt):
    espec = pl.BlockSpec((_BE, _H), lambda i: (i, 0))
    rspec = pl.BlockSpec((_BE, _H), lambda i: ((i + _HBE) % _NBE, 0))
    wspec = pl.BlockSpec((_H, _H), lambda i: (0, 0))
    return pl.pallas_call(
        _gru_body,
        grid=(_NBE,),
        in_specs=[espec, espec, espec, espec, espec, rspec, rspec,
                  wspec, wspec, wspec],
        out_specs=[espec, espec],
        out_shape=[_f32((_E, _H))] * 2,
    )(xz_src, xh_src, xr_dst, g_m, g_rm, m, rm, wz2t, wh2t, urt)


def _final_body(xg, p, wg2, h_o):
    nm = p[0] + p[1]
    h_o[...] = jax.nn.relu(
        xg[...] + jnp.dot(nm, wg2[...], preferred_element_type=jnp.float32))


def _tc_final(xg, p, wg2t):
    return pl.pallas_call(
        _final_body,
        grid=(_N // _BN,),
        in_specs=[pl.BlockSpec((_BN, _H), lambda i: (i, 0)),
                  pl.BlockSpec((2, _BN, _H), lambda i: (0, i, 0)),
                  pl.BlockSpec((_H, _H), lambda i: (0, 0))],
        out_specs=pl.BlockSpec((_BN, _H), lambda i: (i, 0)),
        out_shape=_f32((_N, _H)),
    )(xg, p, wg2t)


def kernel(wid, edge_index, root_ids, emb, Wz_w, Wz_b, Wr_w, Ur_w, Ur_b,
           Wh_w, Wh_b, Wg_w, Wg_b):
    src = edge_index[0].astype(jnp.int32)
    dst = edge_index[1].astype(jnp.int32)
    src32 = _pack_idx(src, 32, 125)
    dst32 = _pack_idx(dst, 32, 125)
    src16 = _pack_idx(src, 16, 250)
    dst16 = _pack_idx(dst, 16, 250)
    widp = _pack_idx(
        jnp.pad(wid.astype(jnp.int32), (0, 10240 - _N)), 32, 4)
    roots = root_ids.astype(jnp.int32)
    zeros_n = jnp.zeros((_NP, _H), jnp.float32)

    wz1t, wz2t = Wz_w[:, :_H].T, Wz_w[:, _H:].T
    wh1t, wh2t = Wh_w[:, :_H].T, Wh_w[:, _H:].T
    wg1t, wg2t = Wg_w[:, :_H].T, Wg_w[:, _H:].T
    wrt, urt = Wr_w.T, Ur_w.T
    bz = Wz_b.reshape(1, _H)
    bh = Wh_b.reshape(1, _H)
    br = Ur_b.reshape(1, _H)
    bg = Wg_b.reshape(1, _H)

    x = _make_gather1(10240, 4, 4)(emb, widp)[:_N]
    xz_n, xh_n, xr_n, xg_n = _tc_nodeproj(x, wz1t, wh1t, wrt, wg1t,
                                          bz, bh, br, bg)

    g1e = _make_gather1(_E, 125, _G)
    xz_src = g1e(xz_n, src32)
    xh_src = g1e(xh_n, src32)
    xr_dst = g1e(xr_n, dst32)

    m, rm = _tc_gru0(xz_src, xh_src, xr_dst, urt)

    segsum2 = _make_segsum2()
    gather2 = _make_gather2()
    for _ in range(3):
        nm, nrm = segsum2(m, rm, dst16, zeros_n)
        g_m, g_rm = gather2(nm, nrm, src16)
        m, rm = _tc_gru(xz_src, xh_src, xr_dst, g_m, g_rm, m, rm,
                        wz2t, wh2t, urt)

    p = _make_segsum_final()(m, dst32, zeros_n)
    h = _tc_final(xg_n, p, wg2t)
    root_vecs = _make_gather_small()(h, roots)
    return (h, root_vecs)


# speedup vs baseline: 3.7496x; 1.1703x over previous
import functools

import jax
import jax.numpy as jnp
from jax import lax
from jax.experimental import pallas as pl
from jax.experimental.pallas import tpu as pltpu
from jax.experimental.pallas import tpu_sc as plsc

_N = 10000
_NP = 10240
_E = 320000
_EU = _E // 2
_H = 128
_C = 80
_G = 5
_GC = _G * _C
_STRIPE = _NP // 16

_mesh = lambda: plsc.VectorSubcoreMesh(core_axis_name="c", subcore_axis_name="s")


def _f32(shape):
    return jax.ShapeDtypeStruct(shape, jnp.float32)


def _ceil8(n):
    return (n + 7) // 8 * 8


def _pack_idx(idx, n_slabs, rows_slab):
    r8 = _ceil8(rows_slab)
    a = idx.reshape(n_slabs, rows_slab, _C)
    a = jnp.pad(a, ((0, 0), (0, r8 - rows_slab), (0, 0)))
    return a.reshape(n_slabs * r8, _C)


def _gather_pipe(table_hbm, idx_v, rows_v, out_hbm, sem_g, sem_st,
                 ebase, n_rows):
    n_groups = n_rows // 2

    def gathers(g, b):
        return [
            pltpu.async_copy(table_hbm.at[idx_v.at[2 * g + j]],
                             rows_v.at[b, pl.ds(j * _C, _C)], sem_g.at[b])
            for j in range(2)
        ]

    def store(g, b):
        return pltpu.async_copy(
            rows_v.at[b], out_hbm.at[pl.ds(ebase + g * 2 * _C, 2 * _C)],
            sem_st.at[b])

    def body(g, carry):
        b = lax.rem(g, 2)

        @pl.when(g >= 2)
        def _():
            pltpu.make_async_copy(
                rows_v.at[b], out_hbm.at[pl.ds(ebase, 2 * _C)],
                sem_st.at[b]).wait()

        gathers(g, b)

        @pl.when(g >= 1)
        def _():
            b1 = lax.rem(g - 1, 2)
            for _j in range(2):
                pltpu.make_async_copy(
                    table_hbm.at[idx_v.at[0]],
                    rows_v.at[b1, pl.ds(_j * _C, _C)], sem_g.at[b1]).wait()
            store(g - 1, b1)

        return carry

    lax.fori_loop(0, n_groups, body, 0)
    bl = (n_groups - 1) % 2
    for _j in range(2):
        pltpu.make_async_copy(table_hbm.at[idx_v.at[0]],
                              rows_v.at[bl, pl.ds(_j * _C, _C)],
                              sem_g.at[bl]).wait()
    store(n_groups - 1, bl)
    pltpu.make_async_copy(rows_v.at[1 - bl],
                          out_hbm.at[pl.ds(ebase, 2 * _C)],
                          sem_st.at[1 - bl]).wait()
    pltpu.make_async_copy(rows_v.at[bl],
                          out_hbm.at[pl.ds(ebase, 2 * _C)],
                          sem_st.at[bl]).wait()


def _make_gather1(n_rows, rows_tile):
    per_tile = rows_tile * _C
    r8 = _ceil8(rows_tile)
    even = rows_tile - (rows_tile % 2)
    assert per_tile * 32 == n_rows

    @functools.partial(
        pl.kernel, mesh=_mesh(),
        out_type=_f32((n_rows, _H)),
        scratch_types=[
            pltpu.VMEM((r8, _C), jnp.int32),
            pltpu.VMEM((2, 2 * _C, _H), jnp.float32),
            pltpu.SemaphoreType.DMA((2,)),
            pltpu.SemaphoreType.DMA((2,)),
        ],
    )
    def k(table_hbm, idx_hbm, out_hbm, idx_v, rows_v, sem_g, sem_st):
        w = lax.axis_index("s") * 2 + lax.axis_index("c")
        pltpu.sync_copy(idx_hbm.at[pl.ds(w * r8, r8)], idx_v)
        _gather_pipe(table_hbm, idx_v, rows_v, out_hbm, sem_g, sem_st,
                     w * per_tile, even)
        if even != rows_tile:
            pltpu.async_copy(table_hbm.at[idx_v.at[rows_tile - 1]],
                             rows_v.at[0, pl.ds(0, _C)], sem_g.at[0]).wait()
            pltpu.sync_copy(rows_v.at[0, pl.ds(0, _C)],
                            out_hbm.at[pl.ds(w * per_tile + even * _C, _C)])

    return k


def _make_gather2():
    rows_tile = 250
    per_tile = rows_tile * _C
    r8 = 256

    @functools.partial(
        pl.kernel, mesh=_mesh(),
        out_type=[_f32((_E, _H)), _f32((_E, _H))],
        scratch_types=[
            pltpu.VMEM((r8, _C), jnp.int32),
            pltpu.VMEM((2, 2 * _C, _H), jnp.float32),
            pltpu.SemaphoreType.DMA((2,)),
            pltpu.SemaphoreType.DMA((2,)),
        ],
    )
    def k(ta, tb, idx_hbm, oa, ob, idx_v, rows_v, sem_g, sem_st):
        cid = lax.axis_index("c")
        sid = lax.axis_index("s")
        pltpu.sync_copy(idx_hbm.at[pl.ds(sid * r8, r8)], idx_v)

        @pl.when(cid == 0)
        def _():
            _gather_pipe(ta, idx_v, rows_v, oa, sem_g, sem_st,
                         sid * per_tile, rows_tile)

        @pl.when(cid == 1)
        def _():
            _gather_pipe(tb, idx_v, rows_v, ob, sem_g, sem_st,
                         sid * per_tile, rows_tile)

    return k


def _make_gather_small():
    @functools.partial(
        pl.kernel, mesh=_mesh(),
        out_type=_f32((32, _H)),
        scratch_types=[
            pltpu.VMEM((32,), jnp.int32),
            pltpu.VMEM((32, _H), jnp.float32),
            pltpu.SemaphoreType.DMA,
        ],
    )
    def k(h_hbm, roots_hbm, out_hbm, idx_v, rows_v, sem):
        cid = lax.axis_index("c")
        sid = lax.axis_index("s")

        @pl.when((cid == 0) & (sid == 0))
        def _():
            pltpu.sync_copy(roots_hbm, idx_v)
            pltpu.async_copy(h_hbm.at[idx_v], rows_v, sem).wait()
            pltpu.sync_copy(rows_v, out_hbm)

    return k


def _scatter_pipe(src_hbm, idx_hbm, idx_base, ebase, n_rows, table,
                  idx_v, rows_v, sem_ld, sem_sc):
    def idx_row(r):
        return idx_v.at[lax.rem(lax.div(r, 32), 2), lax.rem(r, 32)]

    def body(r, carry):
        b = lax.rem(r, 3)

        @pl.when(lax.rem(r, 32) == 0)
        def _():
            sl = lax.div(r, 32)
            pltpu.sync_copy(idx_hbm.at[pl.ds(idx_base + sl * 32, 32)],
                            idx_v.at[lax.rem(sl, 2)])

        @pl.when(r >= 3)
        def _():
            pltpu.make_async_copy(rows_v.at[b], table.at[idx_row(0)],
                                  sem_sc.at[b]).wait()

        pltpu.async_copy(src_hbm.at[pl.ds(ebase + r * _C, _C)],
                         rows_v.at[b], sem_ld.at[b])

        @pl.when(r >= 1)
        def _():
            b1 = lax.rem(r - 1, 3)
            pltpu.make_async_copy(src_hbm.at[pl.ds(ebase, _C)],
                                  rows_v.at[b1], sem_ld.at[b1]).wait()
            pltpu.async_copy(rows_v.at[b1], table.at[idx_row(r - 1)],
                             sem_sc.at[b1], add=True)
        return carry

    lax.fori_loop(0, n_rows, body, 0)
    bl = (n_rows - 1) % 3
    pltpu.make_async_copy(src_hbm.at[pl.ds(ebase, _C)], rows_v.at[bl],
                          sem_ld.at[bl]).wait()
    pltpu.async_copy(rows_v.at[bl], table.at[idx_row(n_rows - 1)],
                     sem_sc.at[bl], add=True)
    for rr in range(max(0, n_rows - 3), n_rows):
        pltpu.make_async_copy(rows_v.at[rr % 3], table.at[idx_row(0)],
                              sem_sc.at[rr % 3]).wait()


def _make_segsum2():
    rows_tile = 250
    r8 = 256

    @functools.partial(
        pl.kernel, mesh=_mesh(),
        out_type=[_f32((_NP, _H)), _f32((_NP, _H))],
        scratch_types=[
            pltpu.VMEM((2, 32, _C), jnp.int32),
            pltpu.VMEM((3, _C, _H), jnp.float32),
            pltpu.VMEM_SHARED((_NP, _H), jnp.float32),
            pltpu.SemaphoreType.DMA((3,)),
            pltpu.SemaphoreType.DMA((3,)),
        ],
    )
    def k(m_hbm, rm_hbm, dst_hbm, zeros_hbm, nm_hbm, nrm_hbm,
          idx_v, rows_v, table, sem_ld, sem_sc):
        cid = lax.axis_index("c")
        sid = lax.axis_index("s")
        pltpu.sync_copy(zeros_hbm.at[pl.ds(sid * _STRIPE, _STRIPE)],
                        table.at[pl.ds(sid * _STRIPE, _STRIPE)])
        plsc.subcore_barrier()

        @pl.when(cid == 0)
        def _():
            _scatter_pipe(m_hbm, dst_hbm, sid * r8, sid * 20000, rows_tile,
                          table, idx_v, rows_v, sem_ld, sem_sc)

        @pl.when(cid == 1)
        def _():
            _scatter_pipe(rm_hbm, dst_hbm, sid * r8, sid * 20000, rows_tile,
                          table, idx_v, rows_v, sem_ld, sem_sc)

        plsc.subcore_barrier()

        @pl.when(cid == 0)
        def _():
            pltpu.sync_copy(table.at[pl.ds(sid * _STRIPE, _STRIPE)],
                            nm_hbm.at[pl.ds(sid * _STRIPE, _STRIPE)])

        @pl.when(cid == 1)
        def _():
            pltpu.sync_copy(table.at[pl.ds(sid * _STRIPE, _STRIPE)],
                            nrm_hbm.at[pl.ds(sid * _STRIPE, _STRIPE)])

    return k


def _make_segsum_final():
    rows_tile = 125
    per_tile = rows_tile * _C
    r8 = 128

    @functools.partial(
        pl.kernel, mesh=_mesh(),
        out_type=_f32((2, _NP, _H)),
        scratch_types=[
            pltpu.VMEM((2, 32, _C), jnp.int32),
            pltpu.VMEM((3, _C, _H), jnp.float32),
            pltpu.VMEM_SHARED((_NP, _H), jnp.float32),
            pltpu.SemaphoreType.DMA((3,)),
            pltpu.SemaphoreType.DMA((3,)),
        ],
    )
    def k(m_hbm, dst_hbm, zeros_hbm, p_hbm, idx_v, rows_v, table,
          sem_ld, sem_sc):
        cid = lax.axis_index("c")
        sid = lax.axis_index("s")
        slab = cid * 16 + sid
        pltpu.sync_copy(zeros_hbm.at[pl.ds(sid * _STRIPE, _STRIPE)],
                        table.at[pl.ds(sid * _STRIPE, _STRIPE)])
        plsc.subcore_barrier()
        _scatter_pipe(m_hbm, dst_hbm, slab * r8, slab * per_tile, rows_tile,
                      table, idx_v, rows_v, sem_ld, sem_sc)
        plsc.subcore_barrier()
        pltpu.sync_copy(table.at[pl.ds(sid * _STRIPE, _STRIPE)],
                        p_hbm.at[cid].at[pl.ds(sid * _STRIPE, _STRIPE)])

    return k


_BN = 1000
_BE = 1600
_NBE = _E // _BE
_HBE = _NBE // 2


def _nodeproj_body(x_ref, wz, wh, wr, wg, bz, bh, br, bg,
                   xz_o, xh_o, xr_o, xg_o):
    xv = x_ref[...]
    f32 = jnp.float32
    xz_o[...] = jnp.dot(xv, wz[...], preferred_element_type=f32) + bz[...]
    xh_o[...] = jnp.dot(xv, wh[...], preferred_element_type=f32) + bh[...]
    xr_o[...] = jnp.dot(xv, wr[...], preferred_element_type=f32) + br[...]
    xg_o[...] = jnp.dot(xv, wg[...], preferred_element_type=f32) + bg[...]


def _tc_nodeproj(x, wz1t, wh1t, wrt, wg1t, bz, bh, br, bg):
    n = x.shape[0]
    wspec = pl.BlockSpec((_H, _H), lambda i: (0, 0))
    bspec = pl.BlockSpec((1, _H), lambda i: (0, 0))
    return pl.pallas_call(
        _nodeproj_body,
        grid=(n // _BN,),
        in_specs=[pl.BlockSpec((_BN, _H), lambda i: (i, 0)),
                  wspec, wspec, wspec, wspec, bspec, bspec, bspec, bspec],
        out_specs=[pl.BlockSpec((_BN, _H), lambda i: (i, 0))] * 4,
        out_shape=[_f32((n, _H))] * 4,
    )(x, wz1t, wh1t, wrt, wg1t, bz, bh, br, bg)


def _gru0_body(xz, xh, xr, ur, m_o, rm_o):
    z = jax.nn.sigmoid(xz[...])
    mn = z * jnp.tanh(xh[...])
    r = jax.nn.sigmoid(
        xr[...] + jnp.dot(mn, ur[...], preferred_element_type=jnp.float32))
    m_o[...] = mn
    rm_o[...] = r * mn


def _tc_gru0(xz_src, xh_src, xr_dst, urt):
    espec = pl.BlockSpec((_BE, _H), lambda i: (i, 0))
    wspec = pl.BlockSpec((_H, _H), lambda i: (0, 0))
    return pl.pallas_call(
        _gru0_body,
        grid=(_NBE,),
        in_specs=[espec, espec, espec, wspec],
        out_specs=[espec, espec],
        out_shape=[_f32((_E, _H))] * 2,
    )(xz_src, xh_src, xr_dst, urt)


def _gru_body(xz, xh, xr, gm, grm, mrev, rmrev, wz2, wh2, ur, m_o, rm_o):
    f32 = jnp.float32
    s = gm[...] - mrev[...]
    a = grm[...] - rmrev[...]
    z = jax.nn.sigmoid(
        xz[...] + jnp.dot(s, wz2[...], preferred_element_type=f32))
    mn = (1.0 - z) * s + z * jnp.tanh(
        xh[...] + jnp.dot(a, wh2[...], preferred_element_type=f32))
    r = jax.nn.sigmoid(
        xr[...] + jnp.dot(mn, ur[...], preferred_element_type=f32))
    m_o[...] = mn
    rm_o[...] = r * mn


def _tc_gru(xz_src, xh_src, xr_dst, g_m, g_rm, m, rm, wz2t, wh2t, urt):
    espec = pl.BlockSpec((_BE, _H), lambda i: (i, 0))
    rspec = pl.BlockSpec((_BE, _H), lambda i: ((i + _HBE) % _NBE, 0))
    wspec = pl.BlockSpec((_H, _H), lambda i: (0, 0))
    return pl.pallas_call(
        _gru_body,
        grid=(_NBE,),
        in_specs=[espec, espec, espec, espec, espec, rspec, rspec,
                  wspec, wspec, wspec],
        out_specs=[espec, espec],
        out_shape=[_f32((_E, _H))] * 2,
    )(xz_src, xh_src, xr_dst, g_m, g_rm, m, rm, wz2t, wh2t, urt)


def _final_body(xg, p, wg2, h_o):
    nm = p[0] + p[1]
    h_o[...] = jax.nn.relu(
        xg[...] + jnp.dot(nm, wg2[...], preferred_element_type=jnp.float32))


def _tc_final(xg, p, wg2t):
    return pl.pallas_call(
        _final_body,
        grid=(_N // _BN,),
        in_specs=[pl.BlockSpec((_BN, _H), lambda i: (i, 0)),
                  pl.BlockSpec((2, _BN, _H), lambda i: (0, i, 0)),
                  pl.BlockSpec((_H, _H), lambda i: (0, 0))],
        out_specs=pl.BlockSpec((_BN, _H), lambda i: (i, 0)),
        out_shape=_f32((_N, _H)),
    )(xg, p, wg2t)


def kernel(wid, edge_index, root_ids, emb, Wz_w, Wz_b, Wr_w, Ur_w, Ur_b,
           Wh_w, Wh_b, Wg_w, Wg_b):
    src = edge_index[0].astype(jnp.int32)
    dst = edge_index[1].astype(jnp.int32)
    src32 = _pack_idx(src, 32, 125)
    dst32 = _pack_idx(dst, 32, 125)
    src16 = _pack_idx(src, 16, 250)
    dst16 = _pack_idx(dst, 16, 250)
    widp = _pack_idx(
        jnp.pad(wid.astype(jnp.int32), (0, 10240 - _N)), 32, 4)
    roots = root_ids.astype(jnp.int32)
    zeros_n = jnp.zeros((_NP, _H), jnp.float32)

    wz1t, wz2t = Wz_w[:, :_H].T, Wz_w[:, _H:].T
    wh1t, wh2t = Wh_w[:, :_H].T, Wh_w[:, _H:].T
    wg1t, wg2t = Wg_w[:, :_H].T, Wg_w[:, _H:].T
    wrt, urt = Wr_w.T, Ur_w.T
    bz = Wz_b.reshape(1, _H)
    bh = Wh_b.reshape(1, _H)
    br = Ur_b.reshape(1, _H)
    bg = Wg_b.reshape(1, _H)

    x = _make_gather1(10240, 4)(emb, widp)[:_N]
    xz_n, xh_n, xr_n, xg_n = _tc_nodeproj(x, wz1t, wh1t, wrt, wg1t,
                                          bz, bh, br, bg)

    gather2 = _make_gather2()
    xz_src, xh_src = gather2(xz_n, xh_n, src16)
    xr_dst = _make_gather1(_E, 125)(xr_n, dst32)

    m, rm = _tc_gru0(xz_src, xh_src, xr_dst, urt)

    segsum2 = _make_segsum2()
    for _ in range(3):
        nm, nrm = segsum2(m, rm, dst16, zeros_n)
        g_m, g_rm = gather2(nm, nrm, src16)
        m, rm = _tc_gru(xz_src, xh_src, xr_dst, g_m, g_rm, m, rm,
                        wz2t, wh2t, urt)

    p = _make_segsum_final()(m, dst32, zeros_n)
    h = _tc_final(xg_n, p, wg2t)
    root_vecs = _make_gather_small()(h, roots)
    return (h, root_vecs)


# speedup vs baseline: 3.7538x; 1.0011x over previous
import functools

import jax
import jax.numpy as jnp
from jax import lax
from jax.experimental import pallas as pl
from jax.experimental.pallas import tpu as pltpu
from jax.experimental.pallas import tpu_sc as plsc

_N = 10000
_NP = 10240
_E = 320000
_EU = _E // 2
_H = 128
_C = 80
_G = 5
_GC = _G * _C
_STRIPE = _NP // 16

_mesh = lambda: plsc.VectorSubcoreMesh(core_axis_name="c", subcore_axis_name="s")


def _f32(shape):
    return jax.ShapeDtypeStruct(shape, jnp.float32)


def _ceil8(n):
    return (n + 7) // 8 * 8


def _pack_idx(idx, n_slabs, rows_slab):
    r8 = _ceil8(rows_slab)
    a = idx.reshape(n_slabs, rows_slab, _C)
    a = jnp.pad(a, ((0, 0), (0, r8 - rows_slab), (0, 0)))
    return a.reshape(n_slabs * r8, _C)


def _gather_pipe(table_hbm, idx_v, rows_v, out_hbm, sem_g, sem_st,
                 ebase, n_rows):
    n_groups = n_rows // 2

    def gathers(g, b):
        return [
            pltpu.async_copy(table_hbm.at[idx_v.at[2 * g + j]],
                             rows_v.at[b, pl.ds(j * _C, _C)], sem_g.at[b])
            for j in range(2)
        ]

    def store(g, b):
        return pltpu.async_copy(
            rows_v.at[b], out_hbm.at[pl.ds(ebase + g * 2 * _C, 2 * _C)],
            sem_st.at[b])

    def body(g, carry):
        b = lax.rem(g, 2)

        @pl.when(g >= 2)
        def _():
            pltpu.make_async_copy(
                rows_v.at[b], out_hbm.at[pl.ds(ebase, 2 * _C)],
                sem_st.at[b]).wait()

        gathers(g, b)

        @pl.when(g >= 1)
        def _():
            b1 = lax.rem(g - 1, 2)
            for _j in range(2):
                pltpu.make_async_copy(
                    table_hbm.at[idx_v.at[0]],
                    rows_v.at[b1, pl.ds(_j * _C, _C)], sem_g.at[b1]).wait()
            store(g - 1, b1)

        return carry

    lax.fori_loop(0, n_groups, body, 0)
    bl = (n_groups - 1) % 2
    for _j in range(2):
        pltpu.make_async_copy(table_hbm.at[idx_v.at[0]],
                              rows_v.at[bl, pl.ds(_j * _C, _C)],
                              sem_g.at[bl]).wait()
    store(n_groups - 1, bl)
    pltpu.make_async_copy(rows_v.at[1 - bl],
                          out_hbm.at[pl.ds(ebase, 2 * _C)],
                          sem_st.at[1 - bl]).wait()
    pltpu.make_async_copy(rows_v.at[bl],
                          out_hbm.at[pl.ds(ebase, 2 * _C)],
                          sem_st.at[bl]).wait()


def _make_gather1(n_rows, rows_tile):
    per_tile = rows_tile * _C
    r8 = _ceil8(rows_tile)
    even = rows_tile - (rows_tile % 2)
    assert per_tile * 32 == n_rows

    @functools.partial(
        pl.kernel, mesh=_mesh(),
        out_type=_f32((n_rows, _H)),
        scratch_types=[
            pltpu.VMEM((r8, _C), jnp.int32),
            pltpu.VMEM((2, 2 * _C, _H), jnp.float32),
            pltpu.SemaphoreType.DMA((2,)),
            pltpu.SemaphoreType.DMA((2,)),
        ],
    )
    def k(table_hbm, idx_hbm, out_hbm, idx_v, rows_v, sem_g, sem_st):
        w = lax.axis_index("s") * 2 + lax.axis_index("c")
        pltpu.sync_copy(idx_hbm.at[pl.ds(w * r8, r8)], idx_v)
        _gather_pipe(table_hbm, idx_v, rows_v, out_hbm, sem_g, sem_st,
                     w * per_tile, even)
        if even != rows_tile:
            pltpu.async_copy(table_hbm.at[idx_v.at[rows_tile - 1]],
                             rows_v.at[0, pl.ds(0, _C)], sem_g.at[0]).wait()
            pltpu.sync_copy(rows_v.at[0, pl.ds(0, _C)],
                            out_hbm.at[pl.ds(w * per_tile + even * _C, _C)])

    return k


def _make_gather2():
    rows_tile = 125
    per_tile = rows_tile * _C
    r8 = 128

    @functools.partial(
        pl.kernel, mesh=_mesh(),
        out_type=[_f32((_E, _H)), _f32((_E, _H))],
        scratch_types=[
            pltpu.VMEM((r8, _C), jnp.int32),
            pltpu.VMEM((2, 2, _C, _H), jnp.float32),
            pltpu.SemaphoreType.DMA((2,)),
            pltpu.SemaphoreType.DMA((2,)),
        ],
    )
    def k(ta, tb, idx_hbm, oa, ob, idx_v, rows_v, sem_g, sem_st):
        w = lax.axis_index("s") * 2 + lax.axis_index("c")
        eb = w * per_tile
        pltpu.sync_copy(idx_hbm.at[pl.ds(w * r8, r8)], idx_v)

        def pair_gather(r, b):
            pltpu.async_copy(ta.at[idx_v.at[r]], rows_v.at[b, 0], sem_g.at[b])
            pltpu.async_copy(tb.at[idx_v.at[r]], rows_v.at[b, 1], sem_g.at[b])

        def pair_store(r, b):
            pltpu.async_copy(rows_v.at[b, 0],
                             oa.at[pl.ds(eb + r * _C, _C)], sem_st.at[b])
            pltpu.async_copy(rows_v.at[b, 1],
                             ob.at[pl.ds(eb + r * _C, _C)], sem_st.at[b])

        def wait_pair(sem, b):
            for t in range(2):
                pltpu.make_async_copy(ta.at[idx_v.at[0]],
                                      rows_v.at[b, t], sem.at[b]).wait()

        def body(r, carry):
            b = lax.rem(r, 2)

            @pl.when(r >= 2)
            def _():
                wait_pair(sem_st, b)

            pair_gather(r, b)

            @pl.when(r >= 1)
            def _():
                b1 = lax.rem(r - 1, 2)
                wait_pair(sem_g, b1)
                pair_store(r - 1, b1)

            return carry

        lax.fori_loop(0, rows_tile, body, 0)
        bl = (rows_tile - 1) % 2
        wait_pair(sem_g, bl)
        pair_store(rows_tile - 1, bl)
        wait_pair(sem_st, 1 - bl)
        wait_pair(sem_st, bl)

    return k


def _make_gather_small():
    @functools.partial(
        pl.kernel, mesh=_mesh(),
        out_type=_f32((32, _H)),
        scratch_types=[
            pltpu.VMEM((32,), jnp.int32),
            pltpu.VMEM((32, _H), jnp.float32),
            pltpu.SemaphoreType.DMA,
        ],
    )
    def k(h_hbm, roots_hbm, out_hbm, idx_v, rows_v, sem):
        cid = lax.axis_index("c")
        sid = lax.axis_index("s")

        @pl.when((cid == 0) & (sid == 0))
        def _():
            pltpu.sync_copy(roots_hbm, idx_v)
            pltpu.async_copy(h_hbm.at[idx_v], rows_v, sem).wait()
            pltpu.sync_copy(rows_v, out_hbm)

    return k


def _scatter_pipe(src_hbm, idx_hbm, idx_base, ebase, n_rows, table,
                  idx_v, rows_v, sem_ld, sem_sc):
    def idx_row(r):
        return idx_v.at[lax.rem(lax.div(r, 32), 2), lax.rem(r, 32)]

    def body(r, carry):
        b = lax.rem(r, 3)

        @pl.when(lax.rem(r, 32) == 0)
        def _():
            sl = lax.div(r, 32)
            pltpu.sync_copy(idx_hbm.at[pl.ds(idx_base + sl * 32, 32)],
                            idx_v.at[lax.rem(sl, 2)])

        @pl.when(r >= 3)
        def _():
            pltpu.make_async_copy(rows_v.at[b], table.at[idx_row(0)],
                                  sem_sc.at[b]).wait()

        pltpu.async_copy(src_hbm.at[pl.ds(ebase + r * _C, _C)],
                         rows_v.at[b], sem_ld.at[b])

        @pl.when(r >= 1)
        def _():
            b1 = lax.rem(r - 1, 3)
            pltpu.make_async_copy(src_hbm.at[pl.ds(ebase, _C)],
                                  rows_v.at[b1], sem_ld.at[b1]).wait()
            pltpu.async_copy(rows_v.at[b1], table.at[idx_row(r - 1)],
                             sem_sc.at[b1], add=True)
        return carry

    lax.fori_loop(0, n_rows, body, 0)
    bl = (n_rows - 1) % 3
    pltpu.make_async_copy(src_hbm.at[pl.ds(ebase, _C)], rows_v.at[bl],
                          sem_ld.at[bl]).wait()
    pltpu.async_copy(rows_v.at[bl], table.at[idx_row(n_rows - 1)],
                     sem_sc.at[bl], add=True)
    for rr in range(max(0, n_rows - 3), n_rows):
        pltpu.make_async_copy(rows_v.at[rr % 3], table.at[idx_row(0)],
                              sem_sc.at[rr % 3]).wait()


def _make_segsum2():
    rows_tile = 250
    r8 = 256

    @functools.partial(
        pl.kernel, mesh=_mesh(),
        out_type=[_f32((_NP, _H)), _f32((_NP, _H))],
        scratch_types=[
            pltpu.VMEM((2, 32, _C), jnp.int32),
            pltpu.VMEM((3, _C, _H), jnp.float32),
            pltpu.VMEM_SHARED((_NP, _H), jnp.float32),
            pltpu.SemaphoreType.DMA((3,)),
            pltpu.SemaphoreType.DMA((3,)),
        ],
    )
    def k(m_hbm, rm_hbm, dst_hbm, zeros_hbm, nm_hbm, nrm_hbm,
          idx_v, rows_v, table, sem_ld, sem_sc):
        cid = lax.axis_index("c")
        sid = lax.axis_index("s")
        pltpu.sync_copy(zeros_hbm.at[pl.ds(sid * _STRIPE, _STRIPE)],
                        table.at[pl.ds(sid * _STRIPE, _STRIPE)])
        plsc.subcore_barrier()

        @pl.when(cid == 0)
        def _():
            _scatter_pipe(m_hbm, dst_hbm, sid * r8, sid * 20000, rows_tile,
                          table, idx_v, rows_v, sem_ld, sem_sc)

        @pl.when(cid == 1)
        def _():
            _scatter_pipe(rm_hbm, dst_hbm, sid * r8, sid * 20000, rows_tile,
                          table, idx_v, rows_v, sem_ld, sem_sc)

        plsc.subcore_barrier()

        @pl.when(cid == 0)
        def _():
            pltpu.sync_copy(table.at[pl.ds(sid * _STRIPE, _STRIPE)],
                            nm_hbm.at[pl.ds(sid * _STRIPE, _STRIPE)])

        @pl.when(cid == 1)
        def _():
            pltpu.sync_copy(table.at[pl.ds(sid * _STRIPE, _STRIPE)],
                            nrm_hbm.at[pl.ds(sid * _STRIPE, _STRIPE)])

    return k


def _make_segsum_final():
    rows_tile = 125
    per_tile = rows_tile * _C
    r8 = 128

    @functools.partial(
        pl.kernel, mesh=_mesh(),
        out_type=_f32((2, _NP, _H)),
        scratch_types=[
            pltpu.VMEM((2, 32, _C), jnp.int32),
            pltpu.VMEM((3, _C, _H), jnp.float32),
            pltpu.VMEM_SHARED((_NP, _H), jnp.float32),
            pltpu.SemaphoreType.DMA((3,)),
            pltpu.SemaphoreType.DMA((3,)),
        ],
    )
    def k(m_hbm, dst_hbm, zeros_hbm, p_hbm, idx_v, rows_v, table,
          sem_ld, sem_sc):
        cid = lax.axis_index("c")
        sid = lax.axis_index("s")
        slab = cid * 16 + sid
        pltpu.sync_copy(zeros_hbm.at[pl.ds(sid * _STRIPE, _STRIPE)],
                        table.at[pl.ds(sid * _STRIPE, _STRIPE)])
        plsc.subcore_barrier()
        _scatter_pipe(m_hbm, dst_hbm, slab * r8, slab * per_tile, rows_tile,
                      table, idx_v, rows_v, sem_ld, sem_sc)
        plsc.subcore_barrier()
        pltpu.sync_copy(table.at[pl.ds(sid * _STRIPE, _STRIPE)],
                        p_hbm.at[cid].at[pl.ds(sid * _STRIPE, _STRIPE)])

    return k


_BN = 1000
_BE = 1600
_NBE = _E // _BE
_HBE = _NBE // 2


def _nodeproj_body(x_ref, wz, wh, wr, wg, bz, bh, br, bg,
                   xz_o, xh_o, xr_o, xg_o):
    xv = x_ref[...]
    f32 = jnp.float32
    xz_o[...] = jnp.dot(xv, wz[...], preferred_element_type=f32) + bz[...]
    xh_o[...] = jnp.dot(xv, wh[...], preferred_element_type=f32) + bh[...]
    xr_o[...] = jnp.dot(xv, wr[...], preferred_element_type=f32) + br[...]
    xg_o[...] = jnp.dot(xv, wg[...], preferred_element_type=f32) + bg[...]


def _tc_nodeproj(x, wz1t, wh1t, wrt, wg1t, bz, bh, br, bg):
    n = x.shape[0]
    wspec = pl.BlockSpec((_H, _H), lambda i: (0, 0))
    bspec = pl.BlockSpec((1, _H), lambda i: (0, 0))
    return pl.pallas_call(
        _nodeproj_body,
        grid=(n // _BN,),
        in_specs=[pl.BlockSpec((_BN, _H), lambda i: (i, 0)),
                  wspec, wspec, wspec, wspec, bspec, bspec, bspec, bspec],
        out_specs=[pl.BlockSpec((_BN, _H), lambda i: (i, 0))] * 4,
        out_shape=[_f32((n, _H))] * 4,
    )(x, wz1t, wh1t, wrt, wg1t, bz, bh, br, bg)


def _gru0_body(xz, xh, xr, ur, m_o, rm_o):
    z = jax.nn.sigmoid(xz[...])
    mn = z * jnp.tanh(xh[...])
    r = jax.nn.sigmoid(
        xr[...] + jnp.dot(mn, ur[...], preferred_element_type=jnp.float32))
    m_o[...] = mn
    rm_o[...] = r * mn


def _tc_gru0(xz_src, xh_src, xr_dst, urt):
    espec = pl.BlockSpec((_BE, _H), lambda i: (i, 0))
    wspec = pl.BlockSpec((_H, _H), lambda i: (0, 0))
    return pl.pallas_call(
        _gru0_body,
        grid=(_NBE,),
        in_specs=[espec, espec, espec, wspec],
        out_specs=[espec, espec],
        out_shape=[_f32((_E, _H))] * 2,
    )(xz_src, xh_src, xr_dst, urt)


def _gru_body(xz, xh, xr, gm, grm, mrev, rmrev, wz2, wh2, ur, m_o, rm_o):
    f32 = jnp.float32
    s = gm[...] - mrev[...]
    a = grm[...] - rmrev[...]
    z = jax.nn.sigmoid(
        xz[...] + jnp.dot(s, wz2[...], preferred_element_type=f32))
    mn = (1.0 - z) * s + z * jnp.tanh(
        xh[...] + jnp.dot(a, wh2[...], preferred_element_type=f32))
    r = jax.nn.sigmoid(
        xr[...] + jnp.dot(mn, ur[...], preferred_element_type=f32))
    m_o[...] = mn
    rm_o[...] = r * mn


def _tc_gru(xz_src, xh_src, xr_dst, g_m, g_rm, m, rm, wz2t, wh2t, urt):
    espec = pl.BlockSpec((_BE, _H), lambda i: (i, 0))
    rspec = pl.BlockSpec((_BE, _H), lambda i: ((i + _HBE) % _NBE, 0))
    wspec = pl.BlockSpec((_H, _H), lambda i: (0, 0))
    return pl.pallas_call(
        _gru_body,
        grid=(_NBE,),
        in_specs=[espec, espec, espec, espec, espec, rspec, rspec,
                  wspec, wspec, wspec],
        out_specs=[espec, espec],
        out_shape=[_f32((_E, _H))] * 2,
    )(xz_src, xh_src, xr_dst, g_m, g_rm, m, rm, wz2t, wh2t, urt)


def _final_body(xg, p, wg2, h_o):
    nm = p[0] + p[1]
    h_o[...] = jax.nn.relu(
        xg[...] + jnp.dot(nm, wg2[...], preferred_element_type=jnp.float32))


def _tc_final(xg, p, wg2t):
    return pl.pallas_call(
        _final_body,
        grid=(_N // _BN,),
        in_specs=[pl.BlockSpec((_BN, _H), lambda i: (i, 0)),
                  pl.BlockSpec((2, _BN, _H), lambda i: (0, i, 0)),
                  pl.BlockSpec((_H, _H), lambda i: (0, 0))],
        out_specs=pl.BlockSpec((_BN, _H), lambda i: (i, 0)),
        out_shape=_f32((_N, _H)),
    )(xg, p, wg2t)


def kernel(wid, edge_index, root_ids, emb, Wz_w, Wz_b, Wr_w, Ur_w, Ur_b,
           Wh_w, Wh_b, Wg_w, Wg_b):
    src = edge_index[0].astype(jnp.int32)
    dst = edge_index[1].astype(jnp.int32)
    src32 = _pack_idx(src, 32, 125)
    dst32 = _pack_idx(dst, 32, 125)
    dst16 = _pack_idx(dst, 16, 250)
    widp = _pack_idx(
        jnp.pad(wid.astype(jnp.int32), (0, 10240 - _N)), 32, 4)
    roots = root_ids.astype(jnp.int32)
    zeros_n = jnp.zeros((_NP, _H), jnp.float32)

    wz1t, wz2t = Wz_w[:, :_H].T, Wz_w[:, _H:].T
    wh1t, wh2t = Wh_w[:, :_H].T, Wh_w[:, _H:].T
    wg1t, wg2t = Wg_w[:, :_H].T, Wg_w[:, _H:].T
    wrt, urt = Wr_w.T, Ur_w.T
    bz = Wz_b.reshape(1, _H)
    bh = Wh_b.reshape(1, _H)
    br = Ur_b.reshape(1, _H)
    bg = Wg_b.reshape(1, _H)

    x = _make_gather1(10240, 4)(emb, widp)[:_N]
    xz_n, xh_n, xr_n, xg_n = _tc_nodeproj(x, wz1t, wh1t, wrt, wg1t,
                                          bz, bh, br, bg)

    gather2 = _make_gather2()
    xz_src, xh_src = gather2(xz_n, xh_n, src32)
    xr_dst = _make_gather1(_E, 125)(xr_n, dst32)

    m, rm = _tc_gru0(xz_src, xh_src, xr_dst, urt)

    segsum2 = _make_segsum2()
    for _ in range(3):
        nm, nrm = segsum2(m, rm, dst16, zeros_n)
        g_m, g_rm = gather2(nm, nrm, src32)
        m, rm = _tc_gru(xz_src, xh_src, xr_dst, g_m, g_rm, m, rm,
                        wz2t, wh2t, urt)

    p = _make_segsum_final()(m, dst32, zeros_n)
    h = _tc_final(xg_n, p, wg2t)
    root_vecs = _make_gather_small()(h, roots)
    return (h, root_vecs)
